# baseline (speedup 1.0000x reference)
import functools

import jax
import jax.numpy as jnp
from jax import lax
from jax.experimental import pallas as pl
from jax.experimental.pallas import tpu as pltpu
from jax.experimental.pallas import tpu_sc as plsc

NEG_SLOPE = 0.2
NC = 2
NS = 16
NW = NC * NS
C = 128


def _proj_body(x_ref, w_ref, a1_ref, a2_ref, h_ref, s1_ref, s2_ref):
    h = jnp.dot(x_ref[...], w_ref[...], preferred_element_type=jnp.float32)
    h_ref[...] = h
    s1_ref[...] = jnp.sum(h * a1_ref[...], axis=1)
    s2_ref[...] = jnp.sum(h * a2_ref[...], axis=1)


def _project(x, w, a1, a2, bm):
    n, f_in = x.shape
    f_out = w.shape[1]
    grid = (n // bm,)
    return pl.pallas_call(
        _proj_body,
        grid=grid,
        in_specs=[
            pl.BlockSpec((bm, f_in), lambda i: (i, 0)),
            pl.BlockSpec((f_in, f_out), lambda i: (0, 0)),
            pl.BlockSpec((1, f_out), lambda i: (0, 0)),
            pl.BlockSpec((1, f_out), lambda i: (0, 0)),
        ],
        out_specs=[
            pl.BlockSpec((bm, f_out), lambda i: (i, 0)),
            pl.BlockSpec((bm,), lambda i: (i,)),
            pl.BlockSpec((bm,), lambda i: (i,)),
        ],
        out_shape=[
            jax.ShapeDtypeStruct((n, f_out), jnp.float32),
            jax.ShapeDtypeStruct((n,), jnp.float32),
            jax.ShapeDtypeStruct((n,), jnp.float32),
        ],
    )(x, w, a1, a2)


def _make_sc(n, f, e_act, chunks, npad):
    fp = f // 2
    pairs = chunks // 2
    rpt = npad // NS
    qcopies = rpt // C

    mesh = plsc.VectorSubcoreMesh(core_axis_name="c", subcore_axis_name="s")

    @functools.partial(
        pl.kernel,
        out_type=[
            jax.ShapeDtypeStruct((NC * npad, f), jnp.float32),
            jax.ShapeDtypeStruct((NC * npad,), jnp.float32),
        ],
        mesh=mesh,
        compiler_params=pltpu.CompilerParams(
            needs_layout_passes=False, use_tc_tiling_on_sc=False),
        scratch_types=[
            pltpu.VMEM_SHARED((npad, f), jnp.float32),
            pltpu.VMEM_SHARED((npad,), jnp.float32),
            pltpu.VMEM((2, C), jnp.int32),
            pltpu.VMEM((2, C), jnp.int32),
            pltpu.VMEM((C,), jnp.float32),
            pltpu.VMEM((C,), jnp.float32),
            pltpu.VMEM((C,), jnp.float32),
            pltpu.VMEM((C,), jnp.float32),
            pltpu.VMEM((C,), jnp.float32),
            pltpu.VMEM((C,), jnp.float32),
            pltpu.VMEM((C, fp), jnp.int32),
            pltpu.VMEM((C, fp), jnp.int32),
            pltpu.VMEM((C, f), jnp.float32),
            pltpu.SemaphoreType.DMA,
            pltpu.SemaphoreType.DMA,
            pltpu.SemaphoreType.DMA,
            pltpu.SemaphoreType.DMA,
        ],
    )
    def sc_kernel(hpk_hbm, s1_hbm, s2_hbm, ij_hbm,
                  acc_out, den_out,
                  acc_sh, den_sh, idx0, idx1, s1c0, s1c1, s2c0, s2c1,
                  ex0, ex1, rp0, rp1, rf, g0, g1, x0, x1):
        cid = lax.axis_index("c")
        sid = lax.axis_index("s")
        wid = cid * NS + sid
        zero16 = jnp.zeros((16,), jnp.float32)

        def zrow(r, carry):
            for fb in range(f // 16):
                rf[r, pl.ds(fb * 16, 16)] = zero16
            return carry
        lax.fori_loop(0, C, zrow, 0)
        for fb in range(C // 16):
            ex0[pl.ds(fb * 16, 16)] = zero16
        base_rows = sid * rpt
        for q in range(qcopies):
            off = pl.multiple_of(base_rows + q * C, 8)
            pltpu.sync_copy(rf, acc_sh.at[pl.ds(off, C)])
            pltpu.sync_copy(ex0, den_sh.at[pl.ds(off, C)])

        idxs = (idx0, idx1)
        s1cs = (s1c0, s1c1)
        s2cs = (s2c0, s2c1)
        exvs = (ex0, ex1)
        rps = (rp0, rp1)
        gsems = (g0, g1)
        xsems = (x0, x1)
        gk0 = wid * chunks

        def start_idx(k, b):
            pltpu.async_copy(ij_hbm.at[gk0 + k], idxs[b], xsems[b])

        def wait_idx(k, b):
            pltpu.make_async_copy(ij_hbm.at[gk0 + k], idxs[b], xsems[b]).wait()

        def start_gathers(b):
            idx = idxs[b]
            pltpu.async_copy(hpk_hbm.at[idx.at[1]], rps[b], gsems[b])
            pltpu.async_copy(s1_hbm.at[idx.at[0]], s1cs[b], gsems[b])
            pltpu.async_copy(s2_hbm.at[idx.at[1]], s2cs[b], gsems[b])

        def wait_gathers(b):
            idx = idxs[b]
            pltpu.make_async_copy(hpk_hbm.at[idx.at[1]], rps[b], gsems[b]).wait()
            pltpu.make_async_copy(s1_hbm.at[idx.at[0]], s1cs[b], gsems[b]).wait()
            pltpu.make_async_copy(s2_hbm.at[idx.at[1]], s2cs[b], gsems[b]).wait()

        start_idx(0, 0)
        start_idx(1, 1)
        wait_idx(0, 0)
        start_gathers(0)
        plsc.subcore_barrier()

        himask = jnp.full((16,), -65536, jnp.int32)

        def process(k, b):
            b2 = 1 - b
            @pl.when(k + 1 < chunks)
            def _():
                wait_idx(k + 1, b2)
                start_gathers(b2)
            idx, exv, rp = idxs[b], exvs[b], rps[b]
            s1c, s2c = s1cs[b], s2cs[b]
            wait_gathers(b)
            ebase_k = (gk0 + k) * C
            for g in range(C // 16):
                sl = pl.ds(g * 16, 16)
                al = s1c[sl] + s2c[sl]
                al = jnp.where(al >= 0.0, al, NEG_SLOPE * al)
                ex = jnp.exp(al)
                eids = ebase_k + g * 16 + lax.iota(jnp.int32, 16)
                ex = jnp.where(eids < e_act, ex, 0.0)
                exv[sl] = ex
            @plsc.parallel_loop(0, C // 16, unroll=1)
            def _(g):
                goff = pl.multiple_of(g * 16, 16)
                ex16 = exv[pl.ds(goff, 16)]
                for l in range(16):
                    s = ex16[l]
                    r = goff + l
                    for cb in range(fp // 16):
                        v = rp[r, pl.ds(cb * 16, 16)]
                        lo = plsc.bitcast(lax.shift_left(v, 16), jnp.float32)
                        hi = plsc.bitcast(v & himask, jnp.float32)
                        rf[r, pl.ds(cb * 16, 16)] = lo * s
                        rf[r, pl.ds(fp + cb * 16, 16)] = hi * s
            pltpu.sync_copy(exv, den_sh.at[idx.at[0]], add=True)
            pltpu.sync_copy(rf, acc_sh.at[idx.at[0]], add=True)
            @pl.when(k + 2 < chunks)
            def _():
                start_idx(k + 2, b)

        def pbody(p, carry):
            k0 = 2 * p
            process(k0, 0)
            process(k0 + 1, 1)
            return carry
        lax.fori_loop(0, pairs, pbody, 0)

        plsc.subcore_barrier()

        woff = pl.multiple_of(cid * npad + base_rows, 8)
        loff = pl.multiple_of(base_rows, 8)
        pltpu.sync_copy(acc_sh.at[pl.ds(loff, rpt)], acc_out.at[pl.ds(woff, rpt)])
        pltpu.sync_copy(den_sh.at[pl.ds(loff, rpt)], den_out.at[pl.ds(woff, rpt)])

    return sc_kernel


def _fin_body(acc_ref, den_ref, bias_ref, out_ref):
    a = acc_ref[0] + acc_ref[1]
    d = den_ref[0] + den_ref[1] + 1e-16
    a = a / d[:, None]
    bf, f = out_ref.shape
    fp = f // 2
    ev = a[:, :fp]
    od = a[:, fp:]
    out_ref[...] = jnp.stack([ev, od], axis=-1).reshape(bf, f) + bias_ref[...]


def _finish(acc, den, bias, bf):
    npad2, f = acc.shape[1], acc.shape[2]
    grid = (npad2 // bf,)
    return pl.pallas_call(
        _fin_body,
        grid=grid,
        in_specs=[
            pl.BlockSpec((2, bf, f), lambda i: (0, i, 0)),
            pl.BlockSpec((2, bf), lambda i: (0, i)),
            pl.BlockSpec((1, f), lambda i: (0, 0)),
        ],
        out_specs=pl.BlockSpec((bf, f), lambda i: (i, 0)),
        out_shape=jax.ShapeDtypeStruct((npad2, f), jnp.float32),
    )(acc, den, bias)


def kernel(x, edge_index, weight, att, bias):
    n, f_in = x.shape
    f = weight.shape[1]
    e = edge_index.shape[1]
    e_act = e + n

    chunks = -(-e_act // (NW * C))
    chunks += chunks % 2
    e_pad = NW * chunks * C
    npad = -(-n // (NS * C)) * (NS * C)

    idt = edge_index.dtype
    loops = jnp.arange(n, dtype=idt)
    padz = jnp.zeros((e_pad - e_act,), dtype=idt)
    ii = jnp.concatenate([edge_index[0], loops, padz])
    jj = jnp.concatenate([edge_index[1], loops, padz])
    ij = jnp.stack([ii.reshape(-1, C), jj.reshape(-1, C)], axis=1)

    a1 = att[0, 0, :f].reshape(1, f)
    a2 = att[0, 0, f:].reshape(1, f)

    x_pad = jnp.concatenate(
        [x, jnp.zeros((npad - n, f_in), dtype=x.dtype)], axis=0)
    h, s1, s2 = _project(x_pad, weight, a1, a2, bm=1024)
    hpk = lax.bitcast_convert_type(
        h.astype(jnp.bfloat16).reshape(npad, f // 2, 2), jnp.int32)

    sc = _make_sc(n, f, e_act, chunks, npad)
    acc_flat, den_flat = sc(hpk, s1, s2, ij)
    acc = acc_flat.reshape(NC, npad, f)
    den = den_flat.reshape(NC, npad)

    out = _finish(acc, den, bias.reshape(1, f), bf=1024)
    return out[:n]


# speedup vs baseline: 1.9650x; 1.9650x over previous
import functools

import jax
import jax.numpy as jnp
from jax import lax
from jax.experimental import pallas as pl
from jax.experimental.pallas import tpu as pltpu
from jax.experimental.pallas import tpu_sc as plsc

NEG_SLOPE = 0.2
NC = 2
NS = 16
C = 128


def _proj_body(x_ref, w_ref, a1_ref, a2_ref, h_ref, s1_ref, s2_ref):
    h = jnp.dot(x_ref[...], w_ref[...], preferred_element_type=jnp.float32)
    h_ref[...] = h
    s1_ref[...] = jnp.sum(h * a1_ref[...], axis=1)
    s2_ref[...] = jnp.sum(h * a2_ref[...], axis=1)


def _project(x, w, a1, a2, bm):
    n, f_in = x.shape
    f_out = w.shape[1]
    grid = (n // bm,)
    return pl.pallas_call(
        _proj_body,
        grid=grid,
        in_specs=[
            pl.BlockSpec((bm, f_in), lambda i: (i, 0)),
            pl.BlockSpec((f_in, f_out), lambda i: (0, 0)),
            pl.BlockSpec((1, f_out), lambda i: (0, 0)),
            pl.BlockSpec((1, f_out), lambda i: (0, 0)),
        ],
        out_specs=[
            pl.BlockSpec((bm, f_out), lambda i: (i, 0)),
            pl.BlockSpec((bm,), lambda i: (i,)),
            pl.BlockSpec((bm,), lambda i: (i,)),
        ],
        out_shape=[
            jax.ShapeDtypeStruct((n, f_out), jnp.float32),
            jax.ShapeDtypeStruct((n,), jnp.float32),
            jax.ShapeDtypeStruct((n,), jnp.float32),
        ],
    )(x, w, a1, a2)


def _make_sc(n, f, e_act, chunks, npad):
    fp = f // 2
    pairs = chunks // 2
    rpt = npad // NS
    qcopies = rpt // C

    mesh = plsc.VectorSubcoreMesh(core_axis_name="c", subcore_axis_name="s")

    @functools.partial(
        pl.kernel,
        out_type=[
            jax.ShapeDtypeStruct((NC * npad, fp), jnp.float32),
            jax.ShapeDtypeStruct((NC * npad,), jnp.float32),
        ],
        mesh=mesh,
        compiler_params=pltpu.CompilerParams(
            needs_layout_passes=False, use_tc_tiling_on_sc=False),
        scratch_types=[
            pltpu.VMEM_SHARED((npad, fp), jnp.float32),
            pltpu.VMEM_SHARED((npad,), jnp.float32),
            pltpu.VMEM_SHARED((npad,), jnp.float32),
            pltpu.VMEM_SHARED((npad, fp), jnp.float32),
            pltpu.VMEM_SHARED((npad,), jnp.float32),
            pltpu.VMEM((2, C), jnp.int32),
            pltpu.VMEM((2, C), jnp.int32),
            pltpu.VMEM((C,), jnp.float32),
            pltpu.VMEM((C,), jnp.float32),
            pltpu.VMEM((C,), jnp.float32),
            pltpu.VMEM((C,), jnp.float32),
            pltpu.VMEM((C,), jnp.float32),
            pltpu.VMEM((C,), jnp.float32),
            pltpu.VMEM((C, fp), jnp.float32),
            pltpu.VMEM((C, fp), jnp.float32),
            pltpu.SemaphoreType.DMA,
            pltpu.SemaphoreType.DMA,
            pltpu.SemaphoreType.DMA,
            pltpu.SemaphoreType.DMA,
        ],
    )
    def sc_kernel(hs_hbm, s1_hbm, s2_hbm, ij_hbm,
                  acc_out, den_out,
                  hsp, s1sp, s2sp, acc_sh, den_sh,
                  idx0, idx1, s1c0, s1c1, s2c0, s2c1,
                  ex0, ex1, r0, r1, g0, g1, x0, x1):
        cid = lax.axis_index("c")
        sid = lax.axis_index("s")
        zero16 = jnp.zeros((16,), jnp.float32)

        def zrow(r, carry):
            for fb in range(fp // 16):
                r0[r, pl.ds(fb * 16, 16)] = zero16
            return carry
        lax.fori_loop(0, C, zrow, 0)
        for fb in range(C // 16):
            ex0[pl.ds(fb * 16, 16)] = zero16
        base_rows = sid * rpt
        for q in range(qcopies):
            off = pl.multiple_of(base_rows + q * C, 8)
            pltpu.sync_copy(r0, acc_sh.at[pl.ds(off, C)])
            pltpu.sync_copy(ex0, den_sh.at[pl.ds(off, C)])

        srow = pl.multiple_of(base_rows, 8)
        pltpu.sync_copy(hs_hbm.at[cid, pl.ds(srow, rpt)], hsp.at[pl.ds(srow, rpt)])
        pltpu.sync_copy(s1_hbm.at[pl.ds(srow, rpt)], s1sp.at[pl.ds(srow, rpt)])
        pltpu.sync_copy(s2_hbm.at[pl.ds(srow, rpt)], s2sp.at[pl.ds(srow, rpt)])

        idxs = (idx0, idx1)
        s1cs = (s1c0, s1c1)
        s2cs = (s2c0, s2c1)
        exvs = (ex0, ex1)
        rows = (r0, r1)
        gsems = (g0, g1)
        xsems = (x0, x1)
        gk0 = sid * chunks

        def start_idx(k, b):
            pltpu.async_copy(ij_hbm.at[gk0 + k], idxs[b], xsems[b])

        def wait_idx(k, b):
            pltpu.make_async_copy(ij_hbm.at[gk0 + k], idxs[b], xsems[b]).wait()

        def start_gathers(b):
            idx = idxs[b]
            pltpu.async_copy(hsp.at[idx.at[1]], rows[b], gsems[b])
            pltpu.async_copy(s1sp.at[idx.at[0]], s1cs[b], gsems[b])
            pltpu.async_copy(s2sp.at[idx.at[1]], s2cs[b], gsems[b])

        def wait_gathers(b):
            idx = idxs[b]
            pltpu.make_async_copy(hsp.at[idx.at[1]], rows[b], gsems[b]).wait()
            pltpu.make_async_copy(s1sp.at[idx.at[0]], s1cs[b], gsems[b]).wait()
            pltpu.make_async_copy(s2sp.at[idx.at[1]], s2cs[b], gsems[b]).wait()

        start_idx(0, 0)
        start_idx(1, 1)
        plsc.subcore_barrier()
        wait_idx(0, 0)
        start_gathers(0)

        def process(k, b):
            b2 = 1 - b
            @pl.when(k + 1 < chunks)
            def _():
                wait_idx(k + 1, b2)
                start_gathers(b2)
            idx, exv, rowsv = idxs[b], exvs[b], rows[b]
            s1c, s2c = s1cs[b], s2cs[b]
            wait_gathers(b)
            ebase_k = (gk0 + k) * C
            for g in range(C // 16):
                sl = pl.ds(g * 16, 16)
                al = s1c[sl] + s2c[sl]
                al = jnp.where(al >= 0.0, al, NEG_SLOPE * al)
                ex = jnp.exp(al)
                eids = ebase_k + g * 16 + lax.iota(jnp.int32, 16)
                ex = jnp.where(eids < e_act, ex, 0.0)
                exv[sl] = ex
            @plsc.parallel_loop(0, C // 16, unroll=1)
            def _(g):
                goff = pl.multiple_of(g * 16, 16)
                ex16 = exv[pl.ds(goff, 16)]
                for l in range(16):
                    s = ex16[l]
                    r = goff + l
                    for fb in range(fp // 16):
                        sl2 = pl.ds(fb * 16, 16)
                        rowsv[r, sl2] = rowsv[r, sl2] * s
            @pl.when(cid == 0)
            def _():
                pltpu.sync_copy(exv, den_sh.at[idx.at[0]], add=True)
            pltpu.sync_copy(rowsv, acc_sh.at[idx.at[0]], add=True)
            @pl.when(k + 2 < chunks)
            def _():
                start_idx(k + 2, b)

        def pbody(p, carry):
            k0 = 2 * p
            process(k0, 0)
            process(k0 + 1, 1)
            return carry
        lax.fori_loop(0, pairs, pbody, 0)

        plsc.subcore_barrier()

        woff = pl.multiple_of(cid * npad + base_rows, 8)
        loff = pl.multiple_of(base_rows, 8)
        pltpu.sync_copy(acc_sh.at[pl.ds(loff, rpt)], acc_out.at[pl.ds(woff, rpt)])
        pltpu.sync_copy(den_sh.at[pl.ds(loff, rpt)], den_out.at[pl.ds(woff, rpt)])

    return sc_kernel


def _fin_body(acc_ref, den_ref, bias_ref, out_ref):
    d = den_ref[0] + den_ref[1] + 1e-16
    a = jnp.concatenate([acc_ref[0], acc_ref[1]], axis=-1)
    out_ref[...] = a / d[:, None] + bias_ref[...]


def _finish(acc, den, bias, bf):
    npad2 = acc.shape[1]
    fp = acc.shape[2]
    f = 2 * fp
    grid = (npad2 // bf,)
    return pl.pallas_call(
        _fin_body,
        grid=grid,
        in_specs=[
            pl.BlockSpec((2, bf, fp), lambda i: (0, i, 0)),
            pl.BlockSpec((2, bf), lambda i: (0, i)),
            pl.BlockSpec((1, f), lambda i: (0, 0)),
        ],
        out_specs=pl.BlockSpec((bf, f), lambda i: (i, 0)),
        out_shape=jax.ShapeDtypeStruct((npad2, f), jnp.float32),
    )(acc, den, bias)


def kernel(x, edge_index, weight, att, bias):
    n, f_in = x.shape
    f = weight.shape[1]
    e = edge_index.shape[1]
    e_act = e + n

    chunks = -(-e_act // (NS * C))
    chunks += chunks % 2
    e_pad = NS * chunks * C
    npad = -(-n // (NS * C)) * (NS * C)

    idt = edge_index.dtype
    loops = jnp.arange(n, dtype=idt)
    padz = jnp.zeros((e_pad - e_act,), dtype=idt)
    ii = jnp.concatenate([edge_index[0], loops, padz])
    jj = jnp.concatenate([edge_index[1], loops, padz])
    ij = jnp.stack([ii.reshape(-1, C), jj.reshape(-1, C)], axis=1)

    a1 = att[0, 0, :f].reshape(1, f)
    a2 = att[0, 0, f:].reshape(1, f)

    x_pad = jnp.concatenate(
        [x, jnp.zeros((npad - n, f_in), dtype=x.dtype)], axis=0)
    h, s1, s2 = _project(x_pad, weight, a1, a2, bm=1024)
    hs = jnp.stack([h[:, :f // 2], h[:, f // 2:]], axis=0)

    sc = _make_sc(n, f, e_act, chunks, npad)
    acc_flat, den_flat = sc(hs, s1, s2, ij)
    acc = acc_flat.reshape(NC, npad, f // 2)
    den = den_flat.reshape(NC, npad)

    out = _finish(acc, den, bias.reshape(1, f), bf=1024)
    return out[:n]


# speedup vs baseline: 2.1436x; 1.0909x over previous
import functools

import jax
import jax.numpy as jnp
from jax import lax
from jax.experimental import pallas as pl
from jax.experimental.pallas import tpu as pltpu
from jax.experimental.pallas import tpu_sc as plsc

NEG_SLOPE = 0.2
NC = 2
NS = 16
C = 128


def _proj_body(x_ref, w_ref, a1_ref, a2_ref, h_ref, s1_ref, s2_ref):
    h = jnp.dot(x_ref[...], w_ref[...], preferred_element_type=jnp.float32)
    h_ref[...] = h
    s1_ref[...] = jnp.sum(h * a1_ref[...], axis=1)
    s2_ref[...] = jnp.sum(h * a2_ref[...], axis=1)


def _project(x, w, a1, a2, bm):
    n, f_in = x.shape
    f_out = w.shape[1]
    grid = (n // bm,)
    return pl.pallas_call(
        _proj_body,
        grid=grid,
        in_specs=[
            pl.BlockSpec((bm, f_in), lambda i: (i, 0)),
            pl.BlockSpec((f_in, f_out), lambda i: (0, 0)),
            pl.BlockSpec((1, f_out), lambda i: (0, 0)),
            pl.BlockSpec((1, f_out), lambda i: (0, 0)),
        ],
        out_specs=[
            pl.BlockSpec((bm, f_out), lambda i: (i, 0)),
            pl.BlockSpec((bm,), lambda i: (i,)),
            pl.BlockSpec((bm,), lambda i: (i,)),
        ],
        out_shape=[
            jax.ShapeDtypeStruct((n, f_out), jnp.float32),
            jax.ShapeDtypeStruct((n,), jnp.float32),
            jax.ShapeDtypeStruct((n,), jnp.float32),
        ],
    )(x, w, a1, a2)


def _make_sc(n, f, e_act, chunks, npad):
    fp = f // 2
    pairs = chunks // 2
    rpt = npad // NS
    qcopies = rpt // C

    mesh = plsc.VectorSubcoreMesh(core_axis_name="c", subcore_axis_name="s")

    @functools.partial(
        pl.kernel,
        out_type=[
            jax.ShapeDtypeStruct((NC * npad, fp), jnp.float32),
            jax.ShapeDtypeStruct((NC * npad,), jnp.float32),
        ],
        mesh=mesh,
        compiler_params=pltpu.CompilerParams(
            needs_layout_passes=False, use_tc_tiling_on_sc=False),
        scratch_types=[
            pltpu.VMEM_SHARED((npad, fp), jnp.float32),
            pltpu.VMEM_SHARED((npad,), jnp.float32),
            pltpu.VMEM_SHARED((npad,), jnp.float32),
            pltpu.VMEM_SHARED((npad, fp), jnp.float32),
            pltpu.VMEM_SHARED((npad,), jnp.float32),
            pltpu.VMEM((2, C), jnp.int32),
            pltpu.VMEM((2, C), jnp.int32),
            pltpu.VMEM((C,), jnp.float32),
            pltpu.VMEM((C,), jnp.float32),
            pltpu.VMEM((C,), jnp.float32),
            pltpu.VMEM((C,), jnp.float32),
            pltpu.VMEM((C,), jnp.float32),
            pltpu.VMEM((C,), jnp.float32),
            pltpu.VMEM((C, fp), jnp.float32),
            pltpu.VMEM((C, fp), jnp.float32),
            pltpu.VMEM((C,), jnp.int32),
            pltpu.VMEM((C,), jnp.int32),
            pltpu.SemaphoreType.DMA,
            pltpu.SemaphoreType.DMA,
            pltpu.SemaphoreType.DMA,
            pltpu.SemaphoreType.DMA,
            pltpu.SemaphoreType.DMA,
            pltpu.SemaphoreType.DMA,
        ],
    )
    def sc_kernel(hs_hbm, s1_hbm, s2_hbm, ij_hbm,
                  acc_out, den_out,
                  hsp, s1sp, s2sp, acc_sh, den_sh,
                  idx0, idx1, s1c0, s1c1, s2c0, s2c1,
                  ex0, ex1, r0, r1, iisc0, iisc1,
                  g0, g1, x0, x1, sc0, sc1):
        cid = lax.axis_index("c")
        sid = lax.axis_index("s")
        zero16 = jnp.zeros((16,), jnp.float32)

        def zrow(r, carry):
            for fb in range(fp // 16):
                r0[r, pl.ds(fb * 16, 16)] = zero16
            return carry
        lax.fori_loop(0, C, zrow, 0)
        for fb in range(C // 16):
            ex0[pl.ds(fb * 16, 16)] = zero16
        base_rows = sid * rpt
        for q in range(qcopies):
            off = pl.multiple_of(base_rows + q * C, 8)
            pltpu.sync_copy(r0, acc_sh.at[pl.ds(off, C)])
            pltpu.sync_copy(ex0, den_sh.at[pl.ds(off, C)])

        srow = pl.multiple_of(base_rows, 8)
        pltpu.sync_copy(hs_hbm.at[cid, pl.ds(srow, rpt)], hsp.at[pl.ds(srow, rpt)])
        pltpu.sync_copy(s1_hbm.at[pl.ds(srow, rpt)], s1sp.at[pl.ds(srow, rpt)])
        pltpu.sync_copy(s2_hbm.at[pl.ds(srow, rpt)], s2sp.at[pl.ds(srow, rpt)])

        idxs = (idx0, idx1)
        s1cs = (s1c0, s1c1)
        s2cs = (s2c0, s2c1)
        exvs = (ex0, ex1)
        rows = (r0, r1)
        iiscs = (iisc0, iisc1)
        gsems = (g0, g1)
        xsems = (x0, x1)
        ssems = (sc0, sc1)
        gk0 = sid * chunks

        def start_idx(k, b):
            pltpu.async_copy(ij_hbm.at[gk0 + k], idxs[b], xsems[b])

        def wait_idx(k, b):
            pltpu.make_async_copy(ij_hbm.at[gk0 + k], idxs[b], xsems[b]).wait()

        def start_gathers(b):
            idx = idxs[b]
            pltpu.async_copy(hsp.at[idx.at[1]], rows[b], gsems[b])
            pltpu.async_copy(s1sp.at[idx.at[0]], s1cs[b], gsems[b])
            pltpu.async_copy(s2sp.at[idx.at[1]], s2cs[b], gsems[b])

        def wait_gathers(b):
            idx = idxs[b]
            pltpu.make_async_copy(hsp.at[idx.at[1]], rows[b], gsems[b]).wait()
            pltpu.make_async_copy(s1sp.at[idx.at[0]], s1cs[b], gsems[b]).wait()
            pltpu.make_async_copy(s2sp.at[idx.at[1]], s2cs[b], gsems[b]).wait()

        start_idx(0, 0)
        start_idx(1, 1)
        plsc.subcore_barrier()
        wait_idx(0, 0)
        start_gathers(0)

        def wait_scatter(b):
            pltpu.make_async_copy(
                rows[b], acc_sh.at[iiscs[b]], ssems[b]).wait()

        def process(k, b):
            b2 = 1 - b
            @pl.when(k + 1 < chunks)
            def _():
                wait_idx(k + 1, b2)

                @pl.when(k >= 1)
                def _():
                    wait_scatter(b2)
                start_gathers(b2)
            idx, exv, rowsv = idxs[b], exvs[b], rows[b]
            s1c, s2c, iisc = s1cs[b], s2cs[b], iiscs[b]
            wait_gathers(b)
            ebase_k = (gk0 + k) * C
            for g in range(C // 16):
                sl = pl.ds(g * 16, 16)
                iisc[sl] = idx[0, sl]
                al = s1c[sl] + s2c[sl]
                al = jnp.where(al >= 0.0, al, NEG_SLOPE * al)
                ex = jnp.exp(al)
                eids = ebase_k + g * 16 + lax.iota(jnp.int32, 16)
                ex = jnp.where(eids < e_act, ex, 0.0)
                exv[sl] = ex
            @pl.when(k + 2 < chunks)
            def _():
                start_idx(k + 2, b)
            @plsc.parallel_loop(0, C // 16, unroll=1)
            def _(g):
                goff = pl.multiple_of(g * 16, 16)
                ex16 = exv[pl.ds(goff, 16)]
                for l in range(16):
                    s = ex16[l]
                    r = goff + l
                    for fb in range(fp // 16):
                        sl2 = pl.ds(fb * 16, 16)
                        rowsv[r, sl2] = rowsv[r, sl2] * s
            @pl.when(cid == b)
            def _():
                pltpu.sync_copy(exv, den_sh.at[iisc], add=True)
            pltpu.async_copy(rowsv, acc_sh.at[iisc], ssems[b], add=True)

        def pbody(p, carry):
            k0 = 2 * p
            process(k0, 0)
            process(k0 + 1, 1)
            return carry
        lax.fori_loop(0, pairs, pbody, 0)

        wait_scatter(0)
        wait_scatter(1)
        plsc.subcore_barrier()

        woff = pl.multiple_of(cid * npad + base_rows, 8)
        loff = pl.multiple_of(base_rows, 8)
        pltpu.sync_copy(acc_sh.at[pl.ds(loff, rpt)], acc_out.at[pl.ds(woff, rpt)])
        pltpu.sync_copy(den_sh.at[pl.ds(loff, rpt)], den_out.at[pl.ds(woff, rpt)])

    return sc_kernel


def _fin_body(acc_ref, den_ref, bias_ref, out_ref):
    d = den_ref[0] + den_ref[1] + 1e-16
    a = jnp.concatenate([acc_ref[0], acc_ref[1]], axis=-1)
    out_ref[...] = a / d[:, None] + bias_ref[...]


def _finish(acc, den, bias, bf):
    npad2 = acc.shape[1]
    fp = acc.shape[2]
    f = 2 * fp
    grid = (npad2 // bf,)
    return pl.pallas_call(
        _fin_body,
        grid=grid,
        in_specs=[
            pl.BlockSpec((2, bf, fp), lambda i: (0, i, 0)),
            pl.BlockSpec((2, bf), lambda i: (0, i)),
            pl.BlockSpec((1, f), lambda i: (0, 0)),
        ],
        out_specs=pl.BlockSpec((bf, f), lambda i: (i, 0)),
        out_shape=jax.ShapeDtypeStruct((npad2, f), jnp.float32),
    )(acc, den, bias)


def kernel(x, edge_index, weight, att, bias):
    n, f_in = x.shape
    f = weight.shape[1]
    e = edge_index.shape[1]
    e_act = e + n

    chunks = -(-e_act // (NS * C))
    chunks += chunks % 2
    e_pad = NS * chunks * C
    npad = -(-n // (NS * C)) * (NS * C)

    idt = edge_index.dtype
    loops = jnp.arange(n, dtype=idt)
    padz = jnp.zeros((e_pad - e_act,), dtype=idt)
    ii = jnp.concatenate([edge_index[0], loops, padz])
    jj = jnp.concatenate([edge_index[1], loops, padz])
    ij = jnp.stack([ii.reshape(-1, C), jj.reshape(-1, C)], axis=1)

    a1 = att[0, 0, :f].reshape(1, f)
    a2 = att[0, 0, f:].reshape(1, f)

    x_pad = jnp.concatenate(
        [x, jnp.zeros((npad - n, f_in), dtype=x.dtype)], axis=0)
    h, s1, s2 = _project(x_pad, weight, a1, a2, bm=1024)
    hs = jnp.stack([h[:, :f // 2], h[:, f // 2:]], axis=0)

    sc = _make_sc(n, f, e_act, chunks, npad)
    acc_flat, den_flat = sc(hs, s1, s2, ij)
    acc = acc_flat.reshape(NC, npad, f // 2)
    den = den_flat.reshape(NC, npad)

    out = _finish(acc, den, bias.reshape(1, f), bf=1024)
    return out[:n]


# speedup vs baseline: 2.3017x; 1.0738x over previous
import functools

import jax
import jax.numpy as jnp
from jax import lax
from jax.experimental import pallas as pl
from jax.experimental.pallas import tpu as pltpu
from jax.experimental.pallas import tpu_sc as plsc

NEG_SLOPE = 0.2
NC = 2
NS = 16
C = 128


def _proj_body(x_ref, w_ref, a1_ref, a2_ref, h_ref, s1_ref, s2_ref):
    h = jnp.dot(x_ref[...], w_ref[...], preferred_element_type=jnp.float32)
    h_ref[...] = h
    s1_ref[...] = jnp.sum(h * a1_ref[...], axis=1)
    s2_ref[...] = jnp.sum(h * a2_ref[...], axis=1)


def _project(x, w, a1, a2, bm):
    n, f_in = x.shape
    f_out = w.shape[1]
    grid = (n // bm,)
    return pl.pallas_call(
        _proj_body,
        grid=grid,
        in_specs=[
            pl.BlockSpec((bm, f_in), lambda i: (i, 0)),
            pl.BlockSpec((f_in, f_out), lambda i: (0, 0)),
            pl.BlockSpec((1, f_out), lambda i: (0, 0)),
            pl.BlockSpec((1, f_out), lambda i: (0, 0)),
        ],
        out_specs=[
            pl.BlockSpec((bm, f_out), lambda i: (i, 0)),
            pl.BlockSpec((bm,), lambda i: (i,)),
            pl.BlockSpec((bm,), lambda i: (i,)),
        ],
        out_shape=[
            jax.ShapeDtypeStruct((n, f_out), jnp.float32),
            jax.ShapeDtypeStruct((n,), jnp.float32),
            jax.ShapeDtypeStruct((n,), jnp.float32),
        ],
    )(x, w, a1, a2)


def _make_sc(n, f, e_act, chunks, npad):
    fp = f // 2
    pairs = chunks // 2
    rpt = npad // NS
    qcopies = rpt // C

    mesh = plsc.VectorSubcoreMesh(core_axis_name="c", subcore_axis_name="s")

    @functools.partial(
        pl.kernel,
        out_type=[
            jax.ShapeDtypeStruct((NC * npad, fp), jnp.float32),
            jax.ShapeDtypeStruct((NC * npad,), jnp.float32),
        ],
        mesh=mesh,
        compiler_params=pltpu.CompilerParams(
            needs_layout_passes=False, use_tc_tiling_on_sc=False),
        scratch_types=[
            pltpu.VMEM_SHARED((npad, fp), jnp.float32),
            pltpu.VMEM_SHARED((npad,), jnp.float32),
            pltpu.VMEM_SHARED((npad,), jnp.float32),
            pltpu.VMEM_SHARED((npad, fp), jnp.float32),
            pltpu.VMEM_SHARED((npad,), jnp.float32),
            pltpu.VMEM((2, C), jnp.int32),
            pltpu.VMEM((2, C), jnp.int32),
            pltpu.VMEM((C,), jnp.float32),
            pltpu.VMEM((C,), jnp.float32),
            pltpu.VMEM((C,), jnp.float32),
            pltpu.VMEM((C,), jnp.float32),
            pltpu.VMEM((C,), jnp.float32),
            pltpu.VMEM((C,), jnp.float32),
            pltpu.VMEM((C, fp), jnp.float32),
            pltpu.VMEM((C, fp), jnp.float32),
            pltpu.VMEM((C,), jnp.int32),
            pltpu.VMEM((C,), jnp.int32),
            pltpu.SemaphoreType.DMA,
            pltpu.SemaphoreType.DMA,
            pltpu.SemaphoreType.DMA,
            pltpu.SemaphoreType.DMA,
            pltpu.SemaphoreType.DMA,
            pltpu.SemaphoreType.DMA,
        ],
    )
    def sc_kernel(hs_hbm, s1_hbm, s2_hbm, ij_hbm,
                  acc_out, den_out,
                  hsp, s1sp, s2sp, acc_sh, den_sh,
                  idx0, idx1, s1c0, s1c1, s2c0, s2c1,
                  ex0, ex1, r0, r1, iisc0, iisc1,
                  g0, g1, x0, x1, sc0, sc1):
        cid = lax.axis_index("c")
        sid = lax.axis_index("s")
        zero16 = jnp.zeros((16,), jnp.float32)

        def zrow(r, carry):
            for fb in range(fp // 16):
                r0[r, pl.ds(fb * 16, 16)] = zero16
            return carry
        lax.fori_loop(0, C, zrow, 0)
        for fb in range(C // 16):
            ex0[pl.ds(fb * 16, 16)] = zero16
        base_rows = sid * rpt
        for q in range(qcopies):
            off = pl.multiple_of(base_rows + q * C, 8)
            pltpu.sync_copy(r0, acc_sh.at[pl.ds(off, C)])
            pltpu.sync_copy(ex0, den_sh.at[pl.ds(off, C)])

        srow = pl.multiple_of(base_rows, 8)
        pltpu.sync_copy(hs_hbm.at[cid, pl.ds(srow, rpt)], hsp.at[pl.ds(srow, rpt)])
        pltpu.sync_copy(s1_hbm.at[pl.ds(srow, rpt)], s1sp.at[pl.ds(srow, rpt)])
        pltpu.sync_copy(s2_hbm.at[pl.ds(srow, rpt)], s2sp.at[pl.ds(srow, rpt)])

        idxs = (idx0, idx1)
        s1cs = (s1c0, s1c1)
        s2cs = (s2c0, s2c1)
        exvs = (ex0, ex1)
        rows = (r0, r1)
        iiscs = (iisc0, iisc1)
        gsems = (g0, g1)
        xsems = (x0, x1)
        ssems = (sc0, sc1)
        gk0 = sid * chunks

        def start_idx(k, b):
            pltpu.async_copy(ij_hbm.at[gk0 + k], idxs[b], xsems[b])

        def wait_idx(k, b):
            pltpu.make_async_copy(ij_hbm.at[gk0 + k], idxs[b], xsems[b]).wait()

        def start_gathers(b):
            idx = idxs[b]
            pltpu.async_copy(hsp.at[idx.at[1]], rows[b], gsems[b])
            pltpu.async_copy(s1sp.at[idx.at[0]], s1cs[b], gsems[b])
            pltpu.async_copy(s2sp.at[idx.at[1]], s2cs[b], gsems[b])

        def wait_gathers(b):
            idx = idxs[b]
            pltpu.make_async_copy(hsp.at[idx.at[1]], rows[b], gsems[b]).wait()
            pltpu.make_async_copy(s1sp.at[idx.at[0]], s1cs[b], gsems[b]).wait()
            pltpu.make_async_copy(s2sp.at[idx.at[1]], s2cs[b], gsems[b]).wait()

        start_idx(0, 0)
        start_idx(1, 1)
        plsc.subcore_barrier()
        wait_idx(0, 0)
        start_gathers(0)

        def wait_scatter(b):
            pltpu.make_async_copy(
                rows[b], acc_sh.at[iiscs[b]], ssems[b]).wait()

        def process(k, b):
            b2 = 1 - b
            @pl.when(k + 1 < chunks)
            def _():
                wait_idx(k + 1, b2)

                @pl.when(k >= 1)
                def _():
                    wait_scatter(b2)
                start_gathers(b2)
            idx, exv, rowsv = idxs[b], exvs[b], rows[b]
            s1c, s2c, iisc = s1cs[b], s2cs[b], iiscs[b]
            wait_gathers(b)
            ebase_k = (gk0 + k) * C
            for g in range(C // 16):
                sl = pl.ds(g * 16, 16)
                iisc[sl] = idx[0, sl]
                al = s1c[sl] + s2c[sl]
                al = jnp.where(al >= 0.0, al, NEG_SLOPE * al)
                ex = jnp.exp(al)
                eids = ebase_k + g * 16 + lax.iota(jnp.int32, 16)
                ex = jnp.where(eids < e_act, ex, 0.0)
                exv[sl] = ex
            @pl.when(k + 2 < chunks)
            def _():
                start_idx(k + 2, b)
            @plsc.parallel_loop(0, C // 16, unroll=2)
            def _(g):
                goff = pl.multiple_of(g * 16, 16)
                ex16 = exv[pl.ds(goff, 16)]
                for l in range(16):
                    s = ex16[l]
                    r = goff + l
                    for fb in range(fp // 16):
                        sl2 = pl.ds(fb * 16, 16)
                        rowsv[r, sl2] = rowsv[r, sl2] * s
            @pl.when(cid == b)
            def _():
                pltpu.sync_copy(exv, den_sh.at[iisc], add=True)
            pltpu.async_copy(rowsv, acc_sh.at[iisc], ssems[b], add=True)

        def pbody(p, carry):
            k0 = 2 * p
            process(k0, 0)
            process(k0 + 1, 1)
            return carry
        lax.fori_loop(0, pairs, pbody, 0)

        wait_scatter(0)
        wait_scatter(1)
        plsc.subcore_barrier()

        woff = pl.multiple_of(cid * npad + base_rows, 8)
        loff = pl.multiple_of(base_rows, 8)
        pltpu.sync_copy(acc_sh.at[pl.ds(loff, rpt)], acc_out.at[pl.ds(woff, rpt)])
        pltpu.sync_copy(den_sh.at[pl.ds(loff, rpt)], den_out.at[pl.ds(woff, rpt)])

    return sc_kernel


def _fin_body(acc_ref, den_ref, bias_ref, out_ref):
    d = den_ref[0] + den_ref[1] + 1e-16
    a = jnp.concatenate([acc_ref[0], acc_ref[1]], axis=-1)
    out_ref[...] = a / d[:, None] + bias_ref[...]


def _finish(acc, den, bias, bf):
    npad2 = acc.shape[1]
    fp = acc.shape[2]
    f = 2 * fp
    grid = (npad2 // bf,)
    return pl.pallas_call(
        _fin_body,
        grid=grid,
        in_specs=[
            pl.BlockSpec((2, bf, fp), lambda i: (0, i, 0)),
            pl.BlockSpec((2, bf), lambda i: (0, i)),
            pl.BlockSpec((1, f), lambda i: (0, 0)),
        ],
        out_specs=pl.BlockSpec((bf, f), lambda i: (i, 0)),
        out_shape=jax.ShapeDtypeStruct((npad2, f), jnp.float32),
    )(acc, den, bias)


def kernel(x, edge_index, weight, att, bias):
    n, f_in = x.shape
    f = weight.shape[1]
    e = edge_index.shape[1]
    e_act = e + n

    chunks = -(-e_act // (NS * C))
    chunks += chunks % 2
    e_pad = NS * chunks * C
    npad = -(-n // (NS * C)) * (NS * C)

    idt = edge_index.dtype
    loops = jnp.arange(n, dtype=idt)
    padz = jnp.zeros((e_pad - e_act,), dtype=idt)
    ii = jnp.concatenate([edge_index[0], loops, padz])
    jj = jnp.concatenate([edge_index[1], loops, padz])
    ij = jnp.stack([ii.reshape(-1, C), jj.reshape(-1, C)], axis=1)

    a1 = att[0, 0, :f].reshape(1, f)
    a2 = att[0, 0, f:].reshape(1, f)

    x_pad = jnp.concatenate(
        [x, jnp.zeros((npad - n, f_in), dtype=x.dtype)], axis=0)
    h, s1, s2 = _project(x_pad, weight, a1, a2, bm=1024)
    hs = jnp.stack([h[:, :f // 2], h[:, f // 2:]], axis=0)

    sc = _make_sc(n, f, e_act, chunks, npad)
    acc_flat, den_flat = sc(hs, s1, s2, ij)
    acc = acc_flat.reshape(NC, npad, f // 2)
    den = den_flat.reshape(NC, npad)

    out = _finish(acc, den, bias.reshape(1, f), bf=1024)
    return out[:n]
